# baseline (speedup 1.0000x reference)
import functools

import jax
import jax.numpy as jnp
from jax import lax
from jax.experimental import pallas as pl
from jax.experimental.pallas import tpu as pltpu
from jax.experimental.pallas import tpu_sc as plsc

B = 4096
L = 200
D = 32
NC = 2
NS = 16
NW = NC * NS
BPW = B // NW
CB = 4
NCHUNK = BPW // CB
LP = 256
G0 = 128
G1 = L - G0


def _sc_bag_sums(x_lin, table):
    mesh = plsc.VectorSubcoreMesh(
        core_axis_name="c", subcore_axis_name="s", num_cores=NC, num_subcores=NS
    )

    @functools.partial(
        pl.kernel,
        out_type=jax.ShapeDtypeStruct((B, D), jnp.float32),
        mesh=mesh,
        compiler_params=pltpu.CompilerParams(use_tc_tiling_on_sc=False),
        scratch_types=[
            pltpu.VMEM((CB * LP,), jnp.int32),
            pltpu.VMEM((CB * LP,), jnp.int32),
            pltpu.VMEM((CB * L, D), jnp.float32),
            pltpu.VMEM((CB * L, D), jnp.float32),
            pltpu.VMEM((BPW, D), jnp.float32),
            pltpu.SemaphoreType.DMA,
            pltpu.SemaphoreType.DMA,
        ],
    )
    def k(x_hbm, tab_hbm, out_hbm, idx0, idx1, rows0, rows1, stage, sem0, sem1):
        wid = lax.axis_index("s") * NC + lax.axis_index("c")
        wbase = wid * BPW

        def fire(g, idx_ref, rows_ref, sem):
            pltpu.sync_copy(
                x_hbm.at[pl.ds((wbase + g * CB) * LP, CB * LP)], idx_ref)
            for bb in range(CB):
                pltpu.async_copy(
                    tab_hbm.at[idx_ref.at[pl.ds(bb * LP, G0)]],
                    rows_ref.at[pl.ds(bb * L, G0), :], sem)
                pltpu.async_copy(
                    tab_hbm.at[idx_ref.at[pl.ds(bb * LP + G0, G1)]],
                    rows_ref.at[pl.ds(bb * L + G0, G1), :], sem)

        def drain(idx_ref, rows_ref, sem):
            for bb in range(CB):
                pltpu.make_async_copy(
                    tab_hbm.at[idx_ref.at[pl.ds(bb * LP, G0)]],
                    rows_ref.at[pl.ds(bb * L, G0), :], sem).wait()
                pltpu.make_async_copy(
                    tab_hbm.at[idx_ref.at[pl.ds(bb * LP + G0, G1)]],
                    rows_ref.at[pl.ds(bb * L + G0, G1), :], sem).wait()

        def accum(g, rows_ref):
            for bb in range(CB):
                base = bb * L
                zz = jnp.zeros((16,), jnp.float32)

                def body(i, carry, base=base, rows_ref=rows_ref):
                    a0, a1, a2, a3, a4, a5, a6, a7 = carry
                    r = base + 4 * i
                    a0 = a0 + rows_ref[r, pl.ds(0, 16)]
                    a1 = a1 + rows_ref[r, pl.ds(16, 16)]
                    a2 = a2 + rows_ref[r + 1, pl.ds(0, 16)]
                    a3 = a3 + rows_ref[r + 1, pl.ds(16, 16)]
                    a4 = a4 + rows_ref[r + 2, pl.ds(0, 16)]
                    a5 = a5 + rows_ref[r + 2, pl.ds(16, 16)]
                    a6 = a6 + rows_ref[r + 3, pl.ds(0, 16)]
                    a7 = a7 + rows_ref[r + 3, pl.ds(16, 16)]
                    return (a0, a1, a2, a3, a4, a5, a6, a7)

                a = lax.fori_loop(0, L // 4, body, (zz,) * 8)
                bag = g * CB + bb
                stage[bag, pl.ds(0, 16)] = (a[0] + a[2]) + (a[4] + a[6])
                stage[bag, pl.ds(16, 16)] = (a[1] + a[3]) + (a[5] + a[7])

        fire(0, idx0, rows0, sem0)

        @pl.loop(0, NCHUNK, step=2)
        def _(g):
            fire(g + 1, idx1, rows1, sem1)
            drain(idx0, rows0, sem0)
            accum(g, rows0)

            @pl.when(g + 2 < NCHUNK)
            def _():
                fire(g + 2, idx0, rows0, sem0)

            drain(idx1, rows1, sem1)
            accum(g + 1, rows1)

        pltpu.sync_copy(stage, out_hbm.at[pl.ds(wbase, BPW), :])

    return k(x_lin, table)


def _tc_head(x, summed, w, b2):
    C = w.shape[0]
    BT = 512

    def body(x_ref, s_ref, w_ref, b_ref, o_ref):
        cnt = jnp.sum((x_ref[...] != 0).astype(jnp.float32), axis=1, keepdims=True)
        denom = jnp.maximum(cnt, 1.0)
        acc = lax.dot_general(
            s_ref[...], w_ref[...], (((1,), (1,)), ((), ())),
            preferred_element_type=jnp.float32,
        )
        o_ref[...] = acc / denom + b_ref[...]

    return pl.pallas_call(
        body,
        grid=(B // BT,),
        in_specs=[
            pl.BlockSpec((BT, L), lambda i: (i, 0)),
            pl.BlockSpec((BT, D), lambda i: (i, 0)),
            pl.BlockSpec((C, D), lambda i: (0, 0)),
            pl.BlockSpec((1, C), lambda i: (0, 0)),
        ],
        out_specs=pl.BlockSpec((BT, C), lambda i: (i, 0)),
        out_shape=jax.ShapeDtypeStruct((B, C), jnp.float32),
    )(x, summed, w, b2)


def kernel(x, table, W, b):
    x = x.astype(jnp.int32)
    x_lin = jnp.pad(x, ((0, 0), (0, LP - L))).reshape(-1)
    summed = _sc_bag_sums(x_lin, table)
    return _tc_head(x, summed, W, b.reshape(1, -1))
